# Initial kernel scaffold; baseline (speedup 1.0000x reference)
import jax
import jax.numpy as jnp
from jax.experimental import pallas as pl


def kernel(rpn_cls_score, rpn_bbox_pred):
    raise NotImplementedError("write your pallas kernel here")


# speedup vs baseline: 14.1926x; 14.1926x over previous
import functools

import jax
import jax.numpy as jnp
import numpy as np
from jax import lax
from jax.experimental import pallas as pl
from jax.experimental.pallas import tpu as pltpu

_FEAT_STRIDE = 16
_H = 50
_W = 50
_A = 9
_N = _H * _W * _A
_R, _C = 176, 128
_PAD = _R * _C - _N
_PRE_TOPN = 6000
_POST_TOPN = 300
_THRESH = 0.7
_IMIN = -2147483648


def _anchor_consts():
    scales = np.array([8.0, 16.0, 32.0])
    ratios = np.array([0.5, 1.0, 2.0])
    base = np.array([1.0, 1.0, 16.0, 16.0]) - 1
    w = base[2] - base[0] + 1
    h = base[3] - base[1] + 1
    x_ctr = base[0] + 0.5 * (w - 1)
    y_ctr = base[1] + 0.5 * (h - 1)
    size_ratios = w * h / ratios
    ws = np.round(np.sqrt(size_ratios))
    hs = np.round(ws * ratios)

    def _mk(ws_, hs_, xc, yc):
        ws_ = ws_[:, None]
        hs_ = hs_[:, None]
        return np.hstack([xc - 0.5 * (ws_ - 1), yc - 0.5 * (hs_ - 1),
                          xc + 0.5 * (ws_ - 1), yc + 0.5 * (hs_ - 1)])

    ratio_anchors = _mk(ws, hs, x_ctr, y_ctr)
    out = []
    for i in range(ratio_anchors.shape[0]):
        a = ratio_anchors[i]
        aw = a[2] - a[0] + 1
        ah = a[3] - a[1] + 1
        axc = a[0] + 0.5 * (aw - 1)
        ayc = a[1] + 0.5 * (ah - 1)
        out.append(_mk(aw * scales, ah * scales, axc, ayc))
    base9 = np.vstack(out).astype(np.float32)

    shift = np.arange(_W, dtype=np.float32) * np.float32(_FEAT_STRIDE)
    sx, sy = np.meshgrid(shift, shift)
    shifts = np.stack([sx.ravel(), sy.ravel(), sx.ravel(), sy.ravel()], axis=1)
    anchors = (shifts[:, None, :].astype(np.float32)
               + base9[None, :, :]).reshape(-1, 4).astype(np.float32)
    x1, y1, x2, y2 = anchors[:, 0], anchors[:, 1], anchors[:, 2], anchors[:, 3]
    wa = x2 - x1 + np.float32(1.0)
    ha = y2 - y1 + np.float32(1.0)
    cxa = x1 + np.float32(0.5) * wa
    cya = y1 + np.float32(0.5) * ha

    def _pad(v):
        return np.pad(v, (0, _PAD)).reshape(_R, _C)

    return _pad(wa), _pad(ha), _pad(cxa), _pad(cya)


_WA, _HA, _CXA, _CYA = _anchor_consts()


def _body(fg_ref, bg_ref, dx_ref, dy_ref, dw_ref, dh_ref,
          wa_ref, ha_ref, cx_ref, cy_ref, out_ref,
          sx1, sy1, sx2, sy2, sar):
    fg = fg_ref[...]
    bg = bg_ref[...]
    wa = wa_ref[...]
    ha = ha_ref[...]

    m = jnp.maximum(fg, bg)
    ef = jnp.exp(fg - m)
    eb = jnp.exp(bg - m)
    sc = ef / (eb + ef)

    pcx = dx_ref[...] * wa + cx_ref[...]
    pcy = dy_ref[...] * ha + cy_ref[...]
    pw = jnp.exp(dw_ref[...]) * wa
    ph = jnp.exp(dh_ref[...]) * ha
    x1 = jnp.clip(pcx - 0.5 * pw, 0.0, 799.0)
    y1 = jnp.clip(pcy - 0.5 * ph, 0.0, 799.0)
    x2 = jnp.clip(pcx + 0.5 * pw, 0.0, 799.0)
    y2 = jnp.clip(pcy + 0.5 * ph, 0.0, 799.0)

    ws = x2 - x1 + 1.0
    hs = y2 - y1 + 1.0
    ok = (ws >= 16.0) & (hs >= 16.0)
    sc = jnp.where(ok, sc, jnp.float32(-1e9))

    ri = lax.broadcasted_iota(jnp.int32, (_R, _C), 0)
    ci = lax.broadcasted_iota(jnp.int32, (_R, _C), 1)
    flat = ri * _C + ci
    sc = jnp.where(flat < _N, sc, jnp.float32(-2e9))

    key = lax.bitcast_convert_type(sc, jnp.int32)
    key = key ^ ((key >> 31) & jnp.int32(0x7FFFFFFF))

    areas = ws * hs
    sx1[...] = x1
    sy1[...] = y1
    sx2[...] = x2
    sy2[...] = y2
    sar[...] = areas

    def bs(_, lh):
        lo, hi = lh
        mid = (lo >> 1) + (hi >> 1) + ((lo | hi) & 1)
        cnt = jnp.sum((key >= mid).astype(jnp.int32))
        p = cnt >= _PRE_TOPN
        return (jnp.where(p, mid, lo), jnp.where(p, hi, mid - 1))

    thr, _ = lax.fori_loop(0, 32, bs, (jnp.int32(_IMIN), jnp.int32(2147483647)))
    alive0 = jnp.where(key >= thr, key, jnp.int32(_IMIN))

    lane8 = lax.broadcasted_iota(jnp.int32, (8, 304), 1)
    row8 = lax.broadcasted_iota(jnp.int32, (8, 304), 0)
    l128 = lax.broadcasted_iota(jnp.int32, (1, _C), 1)

    def nms_body(i, st):
        alive, out = st
        best = jnp.max(alive)
        validb = best > _IMIN
        idx = jnp.min(jnp.where(alive == best, flat, jnp.int32(1 << 30)))
        r = idx // _C
        c = idx % _C

        def pick(ref):
            row = ref[pl.ds(r, 1), :]
            return jnp.max(jnp.where(l128 == c, row, jnp.float32(-3.4e38)))

        bx1 = pick(sx1)
        by1 = pick(sy1)
        bx2 = pick(sx2)
        by2 = pick(sy2)
        bar = pick(sar)

        xx1 = jnp.maximum(bx1, x1)
        yy1 = jnp.maximum(by1, y1)
        xx2 = jnp.minimum(bx2, x2)
        yy2 = jnp.minimum(by2, y2)
        iw = jnp.maximum(0.0, xx2 - xx1 + 1.0)
        ih = jnp.maximum(0.0, yy2 - yy1 + 1.0)
        inter = iw * ih
        iou = inter / (bar + areas - inter)
        alive = jnp.where(validb & (iou > _THRESH), _IMIN, alive)

        vf = jnp.where(validb, jnp.float32(1.0), jnp.float32(0.0))
        vals = jnp.where(row8 == 1, bx1,
               jnp.where(row8 == 2, by1,
               jnp.where(row8 == 3, bx2,
               jnp.where(row8 == 4, by2, jnp.float32(0.0))))) * vf
        out = jnp.where(lane8 == i, vals, out)
        return alive, out

    _, out = lax.fori_loop(
        0, _POST_TOPN, nms_body,
        (alive0, jnp.zeros((8, 304), jnp.float32)))
    out_ref[...] = out


@functools.partial(jax.jit, static_argnames=())
def kernel(rpn_cls_score, rpn_bbox_pred):
    cls = rpn_cls_score.reshape(-1, 2)
    box = rpn_bbox_pred.reshape(-1, 4)

    def prep(v):
        return jnp.pad(v, (0, _PAD)).reshape(_R, _C)

    args = (prep(cls[:, 1]), prep(cls[:, 0]),
            prep(box[:, 0]), prep(box[:, 1]), prep(box[:, 2]), prep(box[:, 3]),
            jnp.asarray(_WA), jnp.asarray(_HA), jnp.asarray(_CXA), jnp.asarray(_CYA))

    out8 = pl.pallas_call(
        _body,
        out_shape=jax.ShapeDtypeStruct((8, 304), jnp.float32),
        scratch_shapes=[pltpu.VMEM((_R, _C), jnp.float32)] * 5,
    )(*args)
    return out8[:5, :_POST_TOPN].T


# speedup vs baseline: 15.6266x; 1.1010x over previous
import functools

import jax
import jax.numpy as jnp
import numpy as np
from jax import lax
from jax.experimental import pallas as pl
from jax.experimental.pallas import tpu as pltpu

_FEAT_STRIDE = 16
_H = 50
_W = 50
_A = 9
_N = _H * _W * _A
_R, _C = 176, 128
_PAD = _R * _C - _N
_PRE_TOPN = 6000
_POST_TOPN = 300
_THRESH = 0.7
_IMIN = -2147483648


def _anchor_consts():
    scales = np.array([8.0, 16.0, 32.0])
    ratios = np.array([0.5, 1.0, 2.0])
    base = np.array([1.0, 1.0, 16.0, 16.0]) - 1
    w = base[2] - base[0] + 1
    h = base[3] - base[1] + 1
    x_ctr = base[0] + 0.5 * (w - 1)
    y_ctr = base[1] + 0.5 * (h - 1)
    size_ratios = w * h / ratios
    ws = np.round(np.sqrt(size_ratios))
    hs = np.round(ws * ratios)

    def _mk(ws_, hs_, xc, yc):
        ws_ = ws_[:, None]
        hs_ = hs_[:, None]
        return np.hstack([xc - 0.5 * (ws_ - 1), yc - 0.5 * (hs_ - 1),
                          xc + 0.5 * (ws_ - 1), yc + 0.5 * (hs_ - 1)])

    ratio_anchors = _mk(ws, hs, x_ctr, y_ctr)
    out = []
    for i in range(ratio_anchors.shape[0]):
        a = ratio_anchors[i]
        aw = a[2] - a[0] + 1
        ah = a[3] - a[1] + 1
        axc = a[0] + 0.5 * (aw - 1)
        ayc = a[1] + 0.5 * (ah - 1)
        out.append(_mk(aw * scales, ah * scales, axc, ayc))
    base9 = np.vstack(out).astype(np.float32)

    shift = np.arange(_W, dtype=np.float32) * np.float32(_FEAT_STRIDE)
    sx, sy = np.meshgrid(shift, shift)
    shifts = np.stack([sx.ravel(), sy.ravel(), sx.ravel(), sy.ravel()], axis=1)
    anchors = (shifts[:, None, :].astype(np.float32)
               + base9[None, :, :]).reshape(-1, 4).astype(np.float32)
    x1, y1, x2, y2 = anchors[:, 0], anchors[:, 1], anchors[:, 2], anchors[:, 3]
    wa = x2 - x1 + np.float32(1.0)
    ha = y2 - y1 + np.float32(1.0)
    cxa = x1 + np.float32(0.5) * wa
    cya = y1 + np.float32(0.5) * ha

    def _pad(v):
        return np.pad(v, (0, _PAD)).reshape(_R, _C)

    return _pad(wa), _pad(ha), _pad(cxa), _pad(cya)


_WA, _HA, _CXA, _CYA = _anchor_consts()


def _body(fg_ref, bg_ref, dx_ref, dy_ref, dw_ref, dh_ref,
          wa_ref, ha_ref, cx_ref, cy_ref, out_ref):
    fg = fg_ref[...]
    bg = bg_ref[...]
    wa = wa_ref[...]
    ha = ha_ref[...]

    m = jnp.maximum(fg, bg)
    ef = jnp.exp(fg - m)
    eb = jnp.exp(bg - m)
    sc = ef / (eb + ef)

    pcx = dx_ref[...] * wa + cx_ref[...]
    pcy = dy_ref[...] * ha + cy_ref[...]
    pw = jnp.exp(dw_ref[...]) * wa
    ph = jnp.exp(dh_ref[...]) * ha
    x1 = jnp.clip(pcx - 0.5 * pw, 0.0, 799.0)
    y1 = jnp.clip(pcy - 0.5 * ph, 0.0, 799.0)
    x2 = jnp.clip(pcx + 0.5 * pw, 0.0, 799.0)
    y2 = jnp.clip(pcy + 0.5 * ph, 0.0, 799.0)

    ws = x2 - x1 + 1.0
    hs = y2 - y1 + 1.0
    ok = (ws >= 16.0) & (hs >= 16.0)
    sc = jnp.where(ok, sc, jnp.float32(-1e9))

    ri = lax.broadcasted_iota(jnp.int32, (_R, _C), 0)
    ci = lax.broadcasted_iota(jnp.int32, (_R, _C), 1)
    flat = ri * _C + ci
    sc = jnp.where(flat < _N, sc, jnp.float32(-2e9))

    key = lax.bitcast_convert_type(sc, jnp.int32)
    key = key ^ ((key >> 31) & jnp.int32(0x7FFFFFFF))

    areas = ws * hs

    def rmax(v):
        return jnp.max(jnp.max(v, axis=0, keepdims=True), axis=1, keepdims=True)

    def rmin(v):
        return jnp.min(jnp.min(v, axis=0, keepdims=True), axis=1, keepdims=True)

    def rsum(v):
        return jnp.sum(jnp.sum(v, axis=0, keepdims=True), axis=1, keepdims=True)

    def bs(_, lh):
        lo, hi = lh
        mid = (lo >> 1) + (hi >> 1) + ((lo | hi) & 1)
        cnt = rsum((key >= mid).astype(jnp.int32))
        p = cnt >= _PRE_TOPN
        return (jnp.where(p, mid, lo), jnp.where(p, hi, mid - 1))

    thr, _ = lax.fori_loop(
        0, 32, bs,
        (jnp.full((1, 1), _IMIN, jnp.int32), jnp.full((1, 1), 2147483647, jnp.int32)))
    alive0 = jnp.where(key >= thr, key, jnp.int32(_IMIN))

    lane8 = lax.broadcasted_iota(jnp.int32, (1, 8), 1)

    def nms_body(i, alive):
        best = rmax(alive)
        validb = best > _IMIN
        eq = alive == best
        fmin = rmin(jnp.where(eq, flat, jnp.int32(1 << 30)))
        onehot = eq & (flat == fmin)

        def pick(v):
            return rmax(jnp.where(onehot, v, jnp.float32(-3.4e38)))

        bx1 = pick(x1)
        by1 = pick(y1)
        bx2 = pick(x2)
        by2 = pick(y2)
        bar = pick(areas)

        xx1 = jnp.maximum(bx1, x1)
        yy1 = jnp.maximum(by1, y1)
        xx2 = jnp.minimum(bx2, x2)
        yy2 = jnp.minimum(by2, y2)
        iw = jnp.maximum(0.0, xx2 - xx1 + 1.0)
        ih = jnp.maximum(0.0, yy2 - yy1 + 1.0)
        inter = iw * ih
        iou = inter / (bar + areas - inter)
        alive = jnp.where(validb & (iou > _THRESH), _IMIN, alive)

        vf = jnp.where(validb, jnp.float32(1.0), jnp.float32(0.0))
        vals = jnp.where(lane8 == 1, bx1,
               jnp.where(lane8 == 2, by1,
               jnp.where(lane8 == 3, bx2,
               jnp.where(lane8 == 4, by2, jnp.float32(0.0))))) * vf
        out_ref[pl.ds(i, 1), :] = vals
        return alive

    lax.fori_loop(0, _POST_TOPN, nms_body, alive0)


@functools.partial(jax.jit, static_argnames=())
def kernel(rpn_cls_score, rpn_bbox_pred):
    cls = rpn_cls_score.reshape(-1, 2)
    box = rpn_bbox_pred.reshape(-1, 4)

    def prep(v):
        return jnp.pad(v, (0, _PAD)).reshape(_R, _C)

    args = (prep(cls[:, 1]), prep(cls[:, 0]),
            prep(box[:, 0]), prep(box[:, 1]), prep(box[:, 2]), prep(box[:, 3]),
            jnp.asarray(_WA), jnp.asarray(_HA), jnp.asarray(_CXA), jnp.asarray(_CYA))

    out8 = pl.pallas_call(
        _body,
        out_shape=jax.ShapeDtypeStruct((304, 8), jnp.float32),
    )(*args)
    return out8[:_POST_TOPN, :5]


# speedup vs baseline: 19.9770x; 1.2784x over previous
import functools

import jax
import jax.numpy as jnp
import numpy as np
from jax import lax
from jax.experimental import pallas as pl
from jax.experimental.pallas import tpu as pltpu

_FEAT_STRIDE = 16
_H = 50
_W = 50
_A = 9
_N = _H * _W * _A
_R, _C = 176, 128
_PAD = _R * _C - _N
_PRE_TOPN = 6000
_POST_TOPN = 300
_THRESH = 0.7
_IMIN = -2147483648


def _anchor_consts():
    scales = np.array([8.0, 16.0, 32.0])
    ratios = np.array([0.5, 1.0, 2.0])
    base = np.array([1.0, 1.0, 16.0, 16.0]) - 1
    w = base[2] - base[0] + 1
    h = base[3] - base[1] + 1
    x_ctr = base[0] + 0.5 * (w - 1)
    y_ctr = base[1] + 0.5 * (h - 1)
    size_ratios = w * h / ratios
    ws = np.round(np.sqrt(size_ratios))
    hs = np.round(ws * ratios)

    def _mk(ws_, hs_, xc, yc):
        ws_ = ws_[:, None]
        hs_ = hs_[:, None]
        return np.hstack([xc - 0.5 * (ws_ - 1), yc - 0.5 * (hs_ - 1),
                          xc + 0.5 * (ws_ - 1), yc + 0.5 * (hs_ - 1)])

    ratio_anchors = _mk(ws, hs, x_ctr, y_ctr)
    out = []
    for i in range(ratio_anchors.shape[0]):
        a = ratio_anchors[i]
        aw = a[2] - a[0] + 1
        ah = a[3] - a[1] + 1
        axc = a[0] + 0.5 * (aw - 1)
        ayc = a[1] + 0.5 * (ah - 1)
        out.append(_mk(aw * scales, ah * scales, axc, ayc))
    base9 = np.vstack(out).astype(np.float32)

    shift = np.arange(_W, dtype=np.float32) * np.float32(_FEAT_STRIDE)
    sx, sy = np.meshgrid(shift, shift)
    shifts = np.stack([sx.ravel(), sy.ravel(), sx.ravel(), sy.ravel()], axis=1)
    anchors = (shifts[:, None, :].astype(np.float32)
               + base9[None, :, :]).reshape(-1, 4).astype(np.float32)
    x1, y1, x2, y2 = anchors[:, 0], anchors[:, 1], anchors[:, 2], anchors[:, 3]
    wa = x2 - x1 + np.float32(1.0)
    ha = y2 - y1 + np.float32(1.0)
    cxa = x1 + np.float32(0.5) * wa
    cya = y1 + np.float32(0.5) * ha

    def _pad(v):
        return np.pad(v, (0, _PAD)).reshape(_R, _C)

    return _pad(wa), _pad(ha), _pad(cxa), _pad(cya)


_WA, _HA, _CXA, _CYA = _anchor_consts()


def _body(fg_ref, bg_ref, dx_ref, dy_ref, dw_ref, dh_ref,
          wa_ref, ha_ref, cx_ref, cy_ref, out_ref):
    fg = fg_ref[...]
    bg = bg_ref[...]
    wa = wa_ref[...]
    ha = ha_ref[...]

    m = jnp.maximum(fg, bg)
    ef = jnp.exp(fg - m)
    eb = jnp.exp(bg - m)
    sc = ef / (eb + ef)

    pcx = dx_ref[...] * wa + cx_ref[...]
    pcy = dy_ref[...] * ha + cy_ref[...]
    pw = jnp.exp(dw_ref[...]) * wa
    ph = jnp.exp(dh_ref[...]) * ha
    x1 = jnp.clip(pcx - 0.5 * pw, 0.0, 799.0)
    y1 = jnp.clip(pcy - 0.5 * ph, 0.0, 799.0)
    x2 = jnp.clip(pcx + 0.5 * pw, 0.0, 799.0)
    y2 = jnp.clip(pcy + 0.5 * ph, 0.0, 799.0)

    ws = x2 - x1 + 1.0
    hs = y2 - y1 + 1.0
    ok = (ws >= 16.0) & (hs >= 16.0)
    sc = jnp.where(ok, sc, jnp.float32(-1e9))

    ri = lax.broadcasted_iota(jnp.int32, (_R, _C), 0)
    ci = lax.broadcasted_iota(jnp.int32, (_R, _C), 1)
    flat = ri * _C + ci
    sc = jnp.where(flat < _N, sc, jnp.float32(-2e9))

    key = lax.bitcast_convert_type(sc, jnp.int32)
    key = key ^ ((key >> 31) & jnp.int32(0x7FFFFFFF))

    areas = ws * hs

    def _tree(v, op):
        chunks = [v[8 * i:8 * (i + 1)] for i in range(_R // 8)]
        while len(chunks) > 1:
            nxt = [op(chunks[2 * j], chunks[2 * j + 1])
                   for j in range(len(chunks) // 2)]
            if len(chunks) % 2:
                nxt.append(chunks[-1])
            chunks = nxt
        r = chunks[0]
        if op is jnp.minimum:
            return jnp.min(jnp.min(r, axis=0, keepdims=True), axis=1, keepdims=True)
        if op is jnp.add:
            return jnp.sum(jnp.sum(r, axis=0, keepdims=True), axis=1, keepdims=True)
        return jnp.max(jnp.max(r, axis=0, keepdims=True), axis=1, keepdims=True)

    def rmax(v):
        return _tree(v, jnp.maximum)

    def rmin(v):
        return _tree(v, jnp.minimum)

    def rsum(v):
        return _tree(v, jnp.add)

    def bs(_, lh):
        lo, hi = lh
        mid = (lo >> 1) + (hi >> 1) + ((lo | hi) & 1)
        cnt = rsum((key >= mid).astype(jnp.int32))
        p = cnt >= _PRE_TOPN
        return (jnp.where(p, mid, lo), jnp.where(p, hi, mid - 1))

    thr, _ = lax.fori_loop(
        0, 32, bs,
        (jnp.full((1, 1), _IMIN, jnp.int32), jnp.full((1, 1), 2147483647, jnp.int32)))
    alive0 = jnp.where(key >= thr, sc, jnp.float32(-3e38))
    flatf = flat.astype(jnp.float32)

    lane8 = lax.broadcasted_iota(jnp.int32, (1, 8), 1)

    def nms_body(i, alive):
        best = rmax(alive)
        validb = best > jnp.float32(-2e9)
        eq = alive == best
        fmin = rmin(jnp.where(eq, flatf, jnp.float32(3e38)))
        onehot = eq & (flatf == fmin)

        def pick(v):
            return rmax(jnp.where(onehot, v, jnp.float32(-3.4e38)))

        bx1 = pick(x1)
        by1 = pick(y1)
        bx2 = pick(x2)
        by2 = pick(y2)
        bar = pick(areas)

        xx1 = jnp.maximum(bx1, x1)
        yy1 = jnp.maximum(by1, y1)
        xx2 = jnp.minimum(bx2, x2)
        yy2 = jnp.minimum(by2, y2)
        iw = jnp.maximum(0.0, xx2 - xx1 + 1.0)
        ih = jnp.maximum(0.0, yy2 - yy1 + 1.0)
        inter = iw * ih
        iou = inter / (bar + areas - inter)
        alive = jnp.where(validb & (iou > _THRESH), jnp.float32(-3e38), alive)

        vf = jnp.where(validb, jnp.float32(1.0), jnp.float32(0.0))
        vals = jnp.where(lane8 == 1, bx1,
               jnp.where(lane8 == 2, by1,
               jnp.where(lane8 == 3, bx2,
               jnp.where(lane8 == 4, by2, jnp.float32(0.0))))) * vf
        out_ref[pl.ds(i, 1), :] = vals
        return alive

    lax.fori_loop(0, _POST_TOPN, nms_body, alive0)


@functools.partial(jax.jit, static_argnames=())
def kernel(rpn_cls_score, rpn_bbox_pred):
    cls = rpn_cls_score.reshape(-1, 2)
    box = rpn_bbox_pred.reshape(-1, 4)

    def prep(v):
        return jnp.pad(v, (0, _PAD)).reshape(_R, _C)

    args = (prep(cls[:, 1]), prep(cls[:, 0]),
            prep(box[:, 0]), prep(box[:, 1]), prep(box[:, 2]), prep(box[:, 3]),
            jnp.asarray(_WA), jnp.asarray(_HA), jnp.asarray(_CXA), jnp.asarray(_CYA))

    out8 = pl.pallas_call(
        _body,
        out_shape=jax.ShapeDtypeStruct((304, 8), jnp.float32),
    )(*args)
    return out8[:_POST_TOPN, :5]
